# baseline (speedup 1.0000x reference)
import jax
import jax.numpy as jnp
from jax import lax
from jax.experimental import pallas as pl
from jax.experimental.pallas import tpu as pltpu
from jax.experimental.pallas import tpu_sc as plsc

B = 128
V = 100000
BSV = 10000
NBV = V // BSV
BIG = 2**30

NC = 2
NS = 16
NW = NC * NS
L = 16
RPW = B // NW


def _tc_shard_argmax(xt_ref, v_ref, i_ref):
    b = pl.program_id(0)
    xb = xt_ref[...]
    m = jnp.max(xb, axis=0, keepdims=True)
    iota = lax.broadcasted_iota(jnp.int32, (BSV, B), 0)
    cand = jnp.where(xb == m, iota, BIG)
    idx = jnp.min(cand, axis=0, keepdims=True) + b * BSV
    v_ref[...] = m.reshape(1, 1, B)
    i_ref[...] = idx.reshape(1, 1, B)


def _sc_merge(v_hbm, i_hbm, out_hbm, vv, ii, res, sem):
    wid = lax.axis_index("s") * NC + lax.axis_index("c")

    pltpu.make_async_copy(v_hbm, vv, sem).start()
    pltpu.make_async_copy(v_hbm, vv, sem).wait()
    pltpu.make_async_copy(i_hbm, ii, sem).start()
    pltpu.make_async_copy(i_hbm, ii, sem).wait()

    lane = lax.iota(jnp.int32, L)
    shard = jnp.minimum(lane, NBV - 1)
    acc = jnp.zeros((L,), jnp.int32)
    for j in range(RPW):
        row = wid * RPW + j
        gidx = shard * B + row
        va = plsc.load_gather(vv, [gidx])
        m = lax.reduce_max(va, axes=(0,))
        ja = plsc.load_gather(ii, [gidx])
        cand = jnp.where(va == m, ja, BIG)
        r = lax.reduce_min(cand, axes=(0,))
        acc = jnp.where(lane == j, r, acc)
    res[...] = acc
    pltpu.sync_copy(res, out_hbm.at[wid])


@jax.jit
def kernel(x):
    xt = jnp.swapaxes(x, 0, 1)
    v2, i2 = pl.pallas_call(
        _tc_shard_argmax,
        grid=(NBV,),
        in_specs=[pl.BlockSpec((BSV, B), lambda b: (b, 0))],
        out_specs=[
            pl.BlockSpec((1, 1, B), lambda b: (b, 0, 0)),
            pl.BlockSpec((1, 1, B), lambda b: (b, 0, 0)),
        ],
        out_shape=[
            jax.ShapeDtypeStruct((NBV, 1, B), jnp.float32),
            jax.ShapeDtypeStruct((NBV, 1, B), jnp.int32),
        ],
    )(xt)

    v1 = v2.reshape(NBV * B)
    i1 = i2.reshape(NBV * B)
    mesh = plsc.VectorSubcoreMesh(core_axis_name="c", subcore_axis_name="s")
    out = pl.kernel(
        _sc_merge,
        out_type=jax.ShapeDtypeStruct((NW, L), jnp.int32),
        mesh=mesh,
        scratch_types=[
            pltpu.VMEM((NBV * B,), jnp.float32),
            pltpu.VMEM((NBV * B,), jnp.int32),
            pltpu.VMEM((L,), jnp.int32),
            pltpu.SemaphoreType.DMA,
        ],
        compiler_params=pltpu.CompilerParams(needs_layout_passes=False),
    )(v1, i1)
    return out[:, :RPW].reshape(B).astype(jnp.int64)
